# baseline (speedup 1.0000x reference)
import functools

import jax
import jax.numpy as jnp
from jax.experimental import pallas as pl
from jax.experimental.pallas import tpu as pltpu

D_MODEL = 1024
HID = 1024
N_ROUTED = 15
TOP_K = 3
EPS = 1e-09
RMS_EPS = 1.1920929e-07

N_TOK = 2048
RT = 256
N_EXP = 16

TM = 512
SH_TILES = N_TOK // TM
G = (N_TOK * TOP_K) // TM + N_ROUTED + SH_TILES
PMAX = G * TM


def _router_body(x_ref, r_ref, xhat_ref, ti_ref, tw_ref):
    x = x_ref[...]
    v = jnp.mean(x * x, axis=-1, keepdims=True)
    xhat_ref[...] = x * jax.lax.rsqrt(v + RMS_EPS)
    logits = jax.lax.dot_general(x, r_ref[...], (((1,), (0,)), ((), ())),
                                 preferred_element_type=jnp.float32)
    m = jnp.max(logits, axis=-1, keepdims=True)
    eg = jnp.exp(logits - m)
    gates = eg / jnp.sum(eg, axis=-1, keepdims=True)
    lanes = jax.lax.broadcasted_iota(jnp.int32, (RT, N_ROUTED), 1)
    g = gates
    idxs, vals = [], []
    for _ in range(TOP_K):
        vals.append(jnp.max(g, axis=-1, keepdims=True))
        j = jnp.argmax(g, axis=-1)[:, None]
        idxs.append(j)
        g = jnp.where(lanes == j, -1.0, g)
    tot = vals[0] + vals[1] + vals[2] + EPS
    topw = jnp.concatenate(vals, axis=1) / tot
    topi = jnp.concatenate(idxs, axis=1)
    ti_ref[...] = jnp.concatenate(
        [topi, jnp.zeros((RT, N_EXP - TOP_K), jnp.int32)], axis=1)
    tw_ref[...] = jnp.concatenate(
        [topw, jnp.zeros((RT, N_EXP - TOP_K), jnp.float32)], axis=1)


def _ffn_body(expert_of_ref, nused_ref, xs_ref, w_ref, W1_ref, W2_ref, ys_ref):
    g = pl.program_id(0)

    @pl.when(g < nused_ref[0])
    def _():
        xh = xs_ref[...]
        h = jax.lax.dot_general(xh, W1_ref[0], (((1,), (1,)), ((), ())),
                                preferred_element_type=jnp.float32)
        h = h * jax.nn.sigmoid(h)
        y = jax.lax.dot_general(h, W2_ref[0], (((1,), (1,)), ((), ())),
                                preferred_element_type=jnp.float32)
        ys_ref[...] = y * w_ref[...]


@jax.jit
def kernel(x, router, W1_r, W2_r, g_r, W1_s, W2_s, g_s):
    B, T, _ = x.shape
    xf = x.reshape(B * T, D_MODEL)
    W1e = jnp.concatenate([W1_r * g_r[:, None, :], W1_s * g_s[:, None, :]], axis=0)
    W2e = jnp.concatenate([W2_r, W2_s], axis=0)

    xhat, ti16, tw16 = pl.pallas_call(
        _router_body,
        grid=(N_TOK // RT,),
        in_specs=[
            pl.BlockSpec((RT, D_MODEL), lambda t: (t, 0)),
            pl.BlockSpec((D_MODEL, N_ROUTED), lambda t: (0, 0)),
        ],
        out_specs=[
            pl.BlockSpec((RT, D_MODEL), lambda t: (t, 0)),
            pl.BlockSpec((RT, N_EXP), lambda t: (t, 0)),
            pl.BlockSpec((RT, N_EXP), lambda t: (t, 0)),
        ],
        out_shape=[
            jax.ShapeDtypeStruct((N_TOK, D_MODEL), jnp.float32),
            jax.ShapeDtypeStruct((N_TOK, N_EXP), jnp.int32),
            jax.ShapeDtypeStruct((N_TOK, N_EXP), jnp.float32),
        ],
    )(xf, router)
    top_i = ti16[:, :TOP_K]
    top_w = tw16[:, :TOP_K]

    onehot = (top_i[:, :, None] == jnp.arange(N_EXP)[None, None, :])
    Xtok = onehot.sum(axis=1).astype(jnp.int32)
    Xc = jnp.cumsum(Xtok, axis=0)
    counts = jnp.where(jnp.arange(N_EXP) == N_EXP - 1, N_TOK, Xc[-1])
    tiles_e = (counts + TM - 1) // TM
    cum_tiles = jnp.cumsum(tiles_e)
    tile_start = cum_tiles - tiles_e
    n_used = cum_tiles[-1]
    pstart = tile_start * TM
    Xex = Xc - Xtok
    rank = jnp.take_along_axis(Xex, top_i, axis=1)
    padpos = pstart[top_i] + rank
    shared_pos = pstart[N_EXP - 1] + jnp.arange(N_TOK)
    comb4 = jnp.concatenate([padpos, shared_pos[:, None]], axis=1)

    tokid = jnp.arange(N_TOK * TOP_K, dtype=jnp.int32) // TOP_K
    tok_full = jnp.zeros((PMAX,), jnp.int32).at[padpos.reshape(-1)].set(tokid)
    w_full = jnp.zeros((PMAX,), jnp.float32).at[padpos.reshape(-1)].set(
        top_w.reshape(-1))
    rows = jnp.arange(PMAX)
    in_shared = (rows >= pstart[N_EXP - 1]) & (rows < pstart[N_EXP - 1] + N_TOK)
    tok_full = jnp.where(in_shared, rows - pstart[N_EXP - 1], tok_full)
    w_full = jnp.where(in_shared, 1.0, w_full)

    expert_of = jnp.minimum(
        jnp.searchsorted(cum_tiles, jnp.arange(G), side="right"),
        N_EXP - 1).astype(jnp.int32)
    nused_arr = jnp.array([0], jnp.int32) + n_used

    xs = xhat[tok_full]

    ys = pl.pallas_call(
        _ffn_body,
        grid_spec=pltpu.PrefetchScalarGridSpec(
            num_scalar_prefetch=2,
            grid=(G,),
            in_specs=[
                pl.BlockSpec((TM, D_MODEL),
                             lambda g, eo, nu: (jnp.minimum(g, nu[0] - 1), 0)),
                pl.BlockSpec((TM, 1),
                             lambda g, eo, nu: (jnp.minimum(g, nu[0] - 1), 0)),
                pl.BlockSpec((1, HID, D_MODEL), lambda g, eo, nu: (eo[g], 0, 0)),
                pl.BlockSpec((1, D_MODEL, HID), lambda g, eo, nu: (eo[g], 0, 0)),
            ],
            out_specs=pl.BlockSpec(
                (TM, D_MODEL), lambda g, eo, nu: (jnp.minimum(g, nu[0] - 1), 0)),
        ),
        out_shape=jax.ShapeDtypeStruct((PMAX, D_MODEL), jnp.float32),
    )(expert_of, nused_arr, xs, w_full[:, None], W1e, W2e)

    out = ys[comb4].sum(axis=1)

    return out.reshape(B, T, D_MODEL)


# speedup vs baseline: 1.2943x; 1.2943x over previous
import functools

import jax
import jax.numpy as jnp
from jax.experimental import pallas as pl
from jax.experimental.pallas import tpu as pltpu

D_MODEL = 1024
HID = 1024
N_ROUTED = 15
TOP_K = 3
EPS = 1e-09
RMS_EPS = 1.1920929e-07

N_TOK = 2048
RT = 256
N_EXP = 16

TM = 512
SH_TILES = N_TOK // TM
G = (N_TOK * TOP_K) // TM + N_ROUTED + SH_TILES
PMAX = G * TM


def _router_body(x_ref, r_ref, xhat_ref, ti_ref, tw_ref):
    x = x_ref[...]
    v = jnp.mean(x * x, axis=-1, keepdims=True)
    xhat_ref[...] = x * jax.lax.rsqrt(v + RMS_EPS)
    logits = jax.lax.dot_general(x, r_ref[...], (((1,), (0,)), ((), ())),
                                 preferred_element_type=jnp.float32)
    m = jnp.max(logits, axis=-1, keepdims=True)
    eg = jnp.exp(logits - m)
    gates = eg / jnp.sum(eg, axis=-1, keepdims=True)
    lanes = jax.lax.broadcasted_iota(jnp.int32, (RT, N_ROUTED), 1)
    g = gates
    idxs, vals = [], []
    for _ in range(TOP_K):
        vals.append(jnp.max(g, axis=-1, keepdims=True))
        j = jnp.argmax(g, axis=-1)[:, None]
        idxs.append(j)
        g = jnp.where(lanes == j, -1.0, g)
    tot = vals[0] + vals[1] + vals[2] + EPS
    topw = jnp.concatenate(vals, axis=1) / tot
    topi = jnp.concatenate(idxs, axis=1)
    ti_ref[...] = jnp.concatenate(
        [topi, jnp.zeros((RT, N_EXP - TOP_K), jnp.int32)], axis=1)
    tw_ref[...] = jnp.concatenate(
        [topw, jnp.zeros((RT, N_EXP - TOP_K), jnp.float32)], axis=1)


def _ffn_body(expert_of_ref, nused_ref, xs_ref, w_ref, W1_ref, W2_ref, ys_ref):
    g = pl.program_id(0)

    @pl.when(g < nused_ref[0])
    def _():
        xh = xs_ref[...]
        h = jax.lax.dot_general(xh, W1_ref[0], (((1,), (1,)), ((), ())),
                                preferred_element_type=jnp.float32)
        h = h * jax.nn.sigmoid(h)
        y = jax.lax.dot_general(h, W2_ref[0], (((1,), (1,)), ((), ())),
                                preferred_element_type=jnp.float32)
        ys_ref[...] = y * w_ref[...]


@jax.jit
def kernel(x, router, W1_r, W2_r, g_r, W1_s, W2_s, g_s):
    B, T, _ = x.shape
    xf = x.reshape(B * T, D_MODEL)
    W1e = jnp.concatenate([W1_r * g_r[:, None, :], W1_s * g_s[:, None, :]], axis=0)
    W2e = jnp.concatenate([W2_r, W2_s], axis=0)

    xhat, ti16, tw16 = pl.pallas_call(
        _router_body,
        grid=(N_TOK // RT,),
        in_specs=[
            pl.BlockSpec((RT, D_MODEL), lambda t: (t, 0)),
            pl.BlockSpec((D_MODEL, N_ROUTED), lambda t: (0, 0)),
        ],
        out_specs=[
            pl.BlockSpec((RT, D_MODEL), lambda t: (t, 0)),
            pl.BlockSpec((RT, N_EXP), lambda t: (t, 0)),
            pl.BlockSpec((RT, N_EXP), lambda t: (t, 0)),
        ],
        out_shape=[
            jax.ShapeDtypeStruct((N_TOK, D_MODEL), jnp.float32),
            jax.ShapeDtypeStruct((N_TOK, N_EXP), jnp.int32),
            jax.ShapeDtypeStruct((N_TOK, N_EXP), jnp.float32),
        ],
    )(xf, router)
    top_i = ti16[:, :TOP_K]
    top_w = tw16[:, :TOP_K]

    onehot = (top_i[:, :, None] == jnp.arange(N_EXP)[None, None, :])
    Xtok = onehot.sum(axis=1).astype(jnp.int32)
    Xc = jnp.cumsum(Xtok, axis=0)
    counts = jnp.where(jnp.arange(N_EXP) == N_EXP - 1, N_TOK, Xc[-1])
    tiles_e = (counts + TM - 1) // TM
    cum_tiles = jnp.cumsum(tiles_e)
    tile_start = cum_tiles - tiles_e
    n_used = cum_tiles[-1]
    pstart = tile_start * TM
    Xex = Xc - Xtok
    rank = jnp.take_along_axis(Xex, top_i, axis=1)
    padpos = pstart[top_i] + rank
    shared_pos = pstart[N_EXP - 1] + jnp.arange(N_TOK)
    comb4 = jnp.concatenate([padpos, shared_pos[:, None]], axis=1)

    tokid = jnp.arange(N_TOK * TOP_K, dtype=jnp.int32) // TOP_K
    tok_full = jnp.zeros((PMAX,), jnp.int32).at[padpos.reshape(-1)].set(tokid)
    w_full = jnp.zeros((PMAX,), jnp.float32).at[padpos.reshape(-1)].set(
        top_w.reshape(-1))
    rows = jnp.arange(PMAX)
    in_shared = (rows >= pstart[N_EXP - 1]) & (rows < pstart[N_EXP - 1] + N_TOK)
    tok_full = jnp.where(in_shared, rows - pstart[N_EXP - 1], tok_full)
    w_full = jnp.where(in_shared, 1.0, w_full)

    expert_of = jnp.minimum(
        jnp.searchsorted(cum_tiles, jnp.arange(G), side="right"),
        N_EXP - 1).astype(jnp.int32)
    nused_arr = jnp.array([0], jnp.int32) + n_used

    xs = jnp.tile(xhat, (8, 1))[:PMAX]

    ys = pl.pallas_call(
        _ffn_body,
        grid_spec=pltpu.PrefetchScalarGridSpec(
            num_scalar_prefetch=2,
            grid=(G,),
            in_specs=[
                pl.BlockSpec((TM, D_MODEL),
                             lambda g, eo, nu: (jnp.minimum(g, nu[0] - 1), 0)),
                pl.BlockSpec((TM, 1),
                             lambda g, eo, nu: (jnp.minimum(g, nu[0] - 1), 0)),
                pl.BlockSpec((1, HID, D_MODEL), lambda g, eo, nu: (eo[g], 0, 0)),
                pl.BlockSpec((1, D_MODEL, HID), lambda g, eo, nu: (eo[g], 0, 0)),
            ],
            out_specs=pl.BlockSpec(
                (TM, D_MODEL), lambda g, eo, nu: (jnp.minimum(g, nu[0] - 1), 0)),
        ),
        out_shape=jax.ShapeDtypeStruct((PMAX, D_MODEL), jnp.float32),
    )(expert_of, nused_arr, xs, w_full[:, None], W1e, W2e)

    out = ys[:N_TOK] + ys[N_TOK:2*N_TOK]

    return out.reshape(B, T, D_MODEL)


# speedup vs baseline: 1.5319x; 1.1836x over previous
import functools

import jax
import jax.numpy as jnp
from jax.experimental import pallas as pl

D_MODEL = 1024
HID = 1024
N_ROUTED = 15
TOP_K = 3
EPS = 1e-09
RMS_EPS = 1.1920929e-07

N_TOK = 2048
RT = 256
FT = 1024
N_EXP = 16


def _router_body(x_ref, r_ref, xhat_ref, w_ref):
    x = x_ref[...]
    v = jnp.mean(x * x, axis=-1, keepdims=True)
    xhat_ref[...] = x * jax.lax.rsqrt(v + RMS_EPS)
    logits = jax.lax.dot_general(x, r_ref[...], (((1,), (0,)), ((), ())),
                                 preferred_element_type=jnp.float32)
    m = jnp.max(logits, axis=-1, keepdims=True)
    eg = jnp.exp(logits - m)
    gates = eg / jnp.sum(eg, axis=-1, keepdims=True)
    lanes = jax.lax.broadcasted_iota(jnp.int32, (RT, N_ROUTED), 1)
    g = gates
    sel = jnp.zeros((RT, N_ROUTED), dtype=jnp.bool_)
    for _ in range(TOP_K):
        j = jnp.argmax(g, axis=-1)[:, None]
        first = lanes == j
        sel = sel | first
        g = jnp.where(first, -1.0, g)
    masked = jnp.where(sel, gates, 0.0)
    w = masked / (jnp.sum(masked, axis=-1, keepdims=True) + EPS)
    w_ref[...] = jnp.concatenate(
        [w, jnp.ones((RT, 1), dtype=jnp.float32)], axis=-1)


def _ffn_body(xhat_ref, w_ref, W1_ref, W2_ref, out_ref):
    e = pl.program_id(1)
    xh = xhat_ref[...].astype(jnp.bfloat16)
    h = jax.lax.dot_general(xh, W1_ref[0].astype(jnp.bfloat16),
                            (((1,), (1,)), ((), ())),
                            preferred_element_type=jnp.float32)
    h = h * jax.nn.sigmoid(h)
    y = jax.lax.dot_general(h.astype(jnp.bfloat16), W2_ref[0].astype(jnp.bfloat16),
                            (((1,), (1,)), ((), ())),
                            preferred_element_type=jnp.float32)
    lanes = jax.lax.broadcasted_iota(jnp.int32, (1, N_EXP), 1)
    wcol = jnp.sum(jnp.where(lanes == e, w_ref[...], 0.0),
                   axis=-1, keepdims=True)
    contrib = y * wcol

    @pl.when(e == 0)
    def _():
        out_ref[...] = contrib

    @pl.when(e != 0)
    def _():
        out_ref[...] += contrib


@jax.jit
def kernel(x, router, W1_r, W2_r, g_r, W1_s, W2_s, g_s):
    B, T, _ = x.shape
    xf = x.reshape(B * T, D_MODEL)
    W1e = jnp.concatenate([W1_r * g_r[:, None, :], W1_s * g_s[:, None, :]], axis=0)
    W2e = jnp.concatenate([W2_r, W2_s], axis=0)

    xhat, w16 = pl.pallas_call(
        _router_body,
        grid=(N_TOK // RT,),
        in_specs=[
            pl.BlockSpec((RT, D_MODEL), lambda t: (t, 0)),
            pl.BlockSpec((D_MODEL, N_ROUTED), lambda t: (0, 0)),
        ],
        out_specs=[
            pl.BlockSpec((RT, D_MODEL), lambda t: (t, 0)),
            pl.BlockSpec((RT, N_EXP), lambda t: (t, 0)),
        ],
        out_shape=[
            jax.ShapeDtypeStruct((N_TOK, D_MODEL), jnp.float32),
            jax.ShapeDtypeStruct((N_TOK, N_EXP), jnp.float32),
        ],
    )(xf, router)

    out = pl.pallas_call(
        _ffn_body,
        grid=(N_TOK // FT, N_EXP),
        in_specs=[
            pl.BlockSpec((FT, D_MODEL), lambda t, e: (t, 0)),
            pl.BlockSpec((FT, N_EXP), lambda t, e: (t, 0)),
            pl.BlockSpec((1, HID, D_MODEL), lambda t, e: (e, 0, 0)),
            pl.BlockSpec((1, D_MODEL, HID), lambda t, e: (e, 0, 0)),
        ],
        out_specs=pl.BlockSpec((FT, D_MODEL), lambda t, e: (t, 0)),
        out_shape=jax.ShapeDtypeStruct((N_TOK, D_MODEL), jnp.float32),
    )(xhat, w16, W1e, W2e)

    return out.reshape(B, T, D_MODEL)


# speedup vs baseline: 1.6502x; 1.0772x over previous
import functools

import jax
import jax.numpy as jnp
from jax.experimental import pallas as pl

D_MODEL = 1024
HID = 1024
N_ROUTED = 15
TOP_K = 3
EPS = 1e-09
RMS_EPS = 1.1920929e-07

N_TOK = 2048
RT = 256
FT = 2048
N_EXP = 16


def _router_body(x_ref, r_ref, xhat_ref, w_ref):
    x = x_ref[...]
    v = jnp.mean(x * x, axis=-1, keepdims=True)
    xhat_ref[...] = (x * jax.lax.rsqrt(v + RMS_EPS)).astype(jnp.bfloat16)
    logits = jax.lax.dot_general(x, r_ref[...], (((1,), (0,)), ((), ())),
                                 preferred_element_type=jnp.float32)
    m = jnp.max(logits, axis=-1, keepdims=True)
    eg = jnp.exp(logits - m)
    gates = eg / jnp.sum(eg, axis=-1, keepdims=True)
    lanes = jax.lax.broadcasted_iota(jnp.int32, (RT, N_ROUTED), 1)
    g = gates
    sel = jnp.zeros((RT, N_ROUTED), dtype=jnp.bool_)
    for _ in range(TOP_K):
        j = jnp.argmax(g, axis=-1)[:, None]
        first = lanes == j
        sel = sel | first
        g = jnp.where(first, -1.0, g)
    masked = jnp.where(sel, gates, 0.0)
    w = masked / (jnp.sum(masked, axis=-1, keepdims=True) + EPS)
    w_ref[...] = jnp.concatenate(
        [w, jnp.ones((RT, 1), dtype=jnp.float32)], axis=-1)


def _ffn_body(xhat_ref, w_ref, W1_ref, W2_ref, out_ref):
    e = pl.program_id(1)
    xh = xhat_ref[...]
    h = jax.lax.dot_general(xh, W1_ref[0], (((1,), (1,)), ((), ())),
                            preferred_element_type=jnp.float32)
    h = h * jax.nn.sigmoid(h)
    y = jax.lax.dot_general(h.astype(jnp.bfloat16), W2_ref[0],
                            (((1,), (1,)), ((), ())),
                            preferred_element_type=jnp.float32)
    lanes = jax.lax.broadcasted_iota(jnp.int32, (1, N_EXP), 1)
    wcol = jnp.sum(jnp.where(lanes == e, w_ref[...], 0.0),
                   axis=-1, keepdims=True)
    contrib = y * wcol

    @pl.when(e == 0)
    def _():
        out_ref[...] = contrib

    @pl.when(e != 0)
    def _():
        out_ref[...] += contrib


@jax.jit
def kernel(x, router, W1_r, W2_r, g_r, W1_s, W2_s, g_s):
    B, T, _ = x.shape
    xf = x.reshape(B * T, D_MODEL)
    W1e = jnp.concatenate([W1_r * g_r[:, None, :], W1_s * g_s[:, None, :]],
                          axis=0).astype(jnp.bfloat16)
    W2e = jnp.concatenate([W2_r, W2_s], axis=0).astype(jnp.bfloat16)

    xhat, w16 = pl.pallas_call(
        _router_body,
        grid=(N_TOK // RT,),
        in_specs=[
            pl.BlockSpec((RT, D_MODEL), lambda t: (t, 0)),
            pl.BlockSpec((D_MODEL, N_ROUTED), lambda t: (0, 0)),
        ],
        out_specs=[
            pl.BlockSpec((RT, D_MODEL), lambda t: (t, 0)),
            pl.BlockSpec((RT, N_EXP), lambda t: (t, 0)),
        ],
        out_shape=[
            jax.ShapeDtypeStruct((N_TOK, D_MODEL), jnp.bfloat16),
            jax.ShapeDtypeStruct((N_TOK, N_EXP), jnp.float32),
        ],
    )(xf, router)

    out = pl.pallas_call(
        _ffn_body,
        grid=(N_TOK // FT, N_EXP),
        in_specs=[
            pl.BlockSpec((FT, D_MODEL), lambda t, e: (t, 0)),
            pl.BlockSpec((FT, N_EXP), lambda t, e: (t, 0)),
            pl.BlockSpec((1, HID, D_MODEL), lambda t, e: (e, 0, 0)),
            pl.BlockSpec((1, D_MODEL, HID), lambda t, e: (e, 0, 0)),
        ],
        out_specs=pl.BlockSpec((FT, D_MODEL), lambda t, e: (t, 0)),
        out_shape=jax.ShapeDtypeStruct((N_TOK, D_MODEL), jnp.float32),
    )(xhat, w16, W1e, W2e)

    return out.reshape(B, T, D_MODEL)


# speedup vs baseline: 1.7666x; 1.0705x over previous
import functools

import jax
import jax.numpy as jnp
from jax import lax
from jax.experimental import pallas as pl
from jax.experimental.pallas import tpu as pltpu
from jax.experimental.pallas import tpu_sc as plsc

D_MODEL = 1024
HID = 1024
N_ROUTED = 15
TOP_K = 3
EPS = 1e-09
RMS_EPS = 1.1920929e-07

N_TOK = 2048
RT = 256
N_EXP = 16

TM = 512
SH_TILES = N_TOK // TM
G = (N_TOK * TOP_K) // TM + N_ROUTED + SH_TILES
PMAX = G * TM

NASSIGN = N_TOK * (TOP_K + 1)

SC_NC = 2
SC_NS = 16
SC_NW = SC_NC * SC_NS
RPW = NASSIGN // SC_NW
CH = 64
NCH = RPW // CH


def _router_body(x_ref, r_ref, xhat_ref, ti_ref, tw_ref):
    x = x_ref[...]
    v = jnp.mean(x * x, axis=-1, keepdims=True)
    xhat_ref[...] = x * jax.lax.rsqrt(v + RMS_EPS)
    logits = jax.lax.dot_general(x, r_ref[...], (((1,), (0,)), ((), ())),
                                 preferred_element_type=jnp.float32)
    m = jnp.max(logits, axis=-1, keepdims=True)
    eg = jnp.exp(logits - m)
    gates = eg / jnp.sum(eg, axis=-1, keepdims=True)
    lanes = jax.lax.broadcasted_iota(jnp.int32, (RT, N_ROUTED), 1)
    g = gates
    idxs, vals = [], []
    for _ in range(TOP_K):
        vals.append(jnp.max(g, axis=-1, keepdims=True))
        j = jnp.argmax(g, axis=-1)[:, None]
        idxs.append(j)
        g = jnp.where(lanes == j, -1.0, g)
    tot = vals[0] + vals[1] + vals[2] + EPS
    topw = jnp.concatenate(vals, axis=1) / tot
    topi = jnp.concatenate(idxs, axis=1)
    ti_ref[...] = jnp.concatenate(
        [topi, jnp.zeros((RT, N_EXP - TOP_K), jnp.int32)], axis=1)
    tw_ref[...] = jnp.concatenate(
        [topw, jnp.zeros((RT, N_EXP - TOP_K), jnp.float32)], axis=1)


def _dispatch_body(xhat_hbm, idx_hbm, xs_hbm, idx_v, rows_v, sem):
    wid = lax.axis_index("s") * SC_NC + lax.axis_index("c")
    pltpu.sync_copy(idx_hbm.at[wid], idx_v)
    base = wid * RPW
    for j in range(NCH):
        src = (base + j * CH) % N_TOK
        pltpu.sync_copy(xhat_hbm.at[pl.ds(src, CH)], rows_v)
        pltpu.async_copy(rows_v, xs_hbm.at[idx_v.at[j]], sem).wait()


def _combine_gather_body(ys_hbm, idx_hbm, ys4_hbm, idx_v, rows_v, sem):
    wid = lax.axis_index("s") * SC_NC + lax.axis_index("c")
    pltpu.sync_copy(idx_hbm.at[wid], idx_v)
    base = wid * RPW
    for j in range(NCH):
        pltpu.async_copy(ys_hbm.at[idx_v.at[j]], rows_v, sem).wait()
        pltpu.sync_copy(rows_v, ys4_hbm.at[pl.ds(base + j * CH, CH)])


def _ffn_body(expert_of_ref, nused_ref, xs_ref, W1_ref, W2_ref, ys_ref):
    g = pl.program_id(0)

    @pl.when(g < nused_ref[0])
    def _():
        xh = xs_ref[...].astype(jnp.bfloat16)
        h = jax.lax.dot_general(xh, W1_ref[0], (((1,), (1,)), ((), ())),
                                preferred_element_type=jnp.float32)
        h = h * jax.nn.sigmoid(h)
        y = jax.lax.dot_general(h.astype(jnp.bfloat16), W2_ref[0],
                                (((1,), (1,)), ((), ())),
                                preferred_element_type=jnp.float32)
        ys_ref[...] = y


def _combine_body(ys4_ref, w4_ref, out_ref):
    acc = ys4_ref[0] * w4_ref[0, :][:, None]
    for k in range(1, TOP_K + 1):
        acc += ys4_ref[k] * w4_ref[k, :][:, None]
    out_ref[...] = acc


@jax.jit
def kernel(x, router, W1_r, W2_r, g_r, W1_s, W2_s, g_s):
    B, T, _ = x.shape
    xf = x.reshape(B * T, D_MODEL)
    W1e = jnp.concatenate([W1_r * g_r[:, None, :], W1_s * g_s[:, None, :]],
                          axis=0).astype(jnp.bfloat16)
    W2e = jnp.concatenate([W2_r, W2_s], axis=0).astype(jnp.bfloat16)

    xhat, ti16, tw16 = pl.pallas_call(
        _router_body,
        grid=(N_TOK // RT,),
        in_specs=[
            pl.BlockSpec((RT, D_MODEL), lambda t: (t, 0)),
            pl.BlockSpec((D_MODEL, N_ROUTED), lambda t: (0, 0)),
        ],
        out_specs=[
            pl.BlockSpec((RT, D_MODEL), lambda t: (t, 0)),
            pl.BlockSpec((RT, N_EXP), lambda t: (t, 0)),
            pl.BlockSpec((RT, N_EXP), lambda t: (t, 0)),
        ],
        out_shape=[
            jax.ShapeDtypeStruct((N_TOK, D_MODEL), jnp.float32),
            jax.ShapeDtypeStruct((N_TOK, N_EXP), jnp.int32),
            jax.ShapeDtypeStruct((N_TOK, N_EXP), jnp.float32),
        ],
    )(xf, router)
    top_i = ti16[:, :TOP_K]
    top_w = tw16[:, :TOP_K]

    e_lanes = jnp.arange(N_EXP, dtype=jnp.int32)
    onehot = (top_i[:, :, None] == e_lanes[None, None, :]).astype(jnp.int32)
    Xtok = onehot.sum(axis=1)
    Xc = jnp.cumsum(Xtok, axis=0)
    counts = jnp.where(e_lanes == N_EXP - 1, N_TOK, Xc[-1])
    tiles_e = (counts + TM - 1) // TM
    cum_tiles = jnp.cumsum(tiles_e)
    tile_start = cum_tiles - tiles_e
    n_used = cum_tiles[-1]
    pstart = tile_start * TM
    Xex = Xc - Xtok
    rank = (onehot * Xex[:, None, :]).sum(-1)
    padpos = (onehot * pstart[None, None, :]).sum(-1) + rank
    shared_pos = pstart[N_EXP - 1] + jnp.arange(N_TOK, dtype=jnp.int32)
    padflat = jnp.concatenate(
        [padpos[:, 0], padpos[:, 1], padpos[:, 2], shared_pos])
    idx3d = padflat.reshape(SC_NW, NCH, CH)
    w4 = jnp.concatenate(
        [top_w.T, jnp.ones((1, N_TOK), jnp.float32)], axis=0)

    grange = jnp.arange(G, dtype=jnp.int32)
    expert_of = jnp.minimum(
        (grange[:, None] >= cum_tiles[None, :]).astype(jnp.int32).sum(-1),
        N_EXP - 1)
    nused_arr = n_used[None]

    mesh = plsc.VectorSubcoreMesh(core_axis_name="c", subcore_axis_name="s")
    xs = pl.kernel(
        _dispatch_body, mesh=mesh,
        out_type=jax.ShapeDtypeStruct((PMAX, D_MODEL), jnp.float32),
        scratch_types=[
            pltpu.VMEM((NCH, CH), jnp.int32),
            pltpu.VMEM((CH, D_MODEL), jnp.float32),
            pltpu.SemaphoreType.DMA,
        ],
    )(xhat, idx3d)

    ys = pl.pallas_call(
        _ffn_body,
        grid_spec=pltpu.PrefetchScalarGridSpec(
            num_scalar_prefetch=2,
            grid=(G,),
            in_specs=[
                pl.BlockSpec((TM, D_MODEL),
                             lambda g, eo, nu: (jnp.minimum(g, nu[0] - 1), 0)),
                pl.BlockSpec((1, HID, D_MODEL), lambda g, eo, nu: (eo[g], 0, 0)),
                pl.BlockSpec((1, D_MODEL, HID), lambda g, eo, nu: (eo[g], 0, 0)),
            ],
            out_specs=pl.BlockSpec(
                (TM, D_MODEL), lambda g, eo, nu: (jnp.minimum(g, nu[0] - 1), 0)),
        ),
        out_shape=jax.ShapeDtypeStruct((PMAX, D_MODEL), jnp.float32),
    )(expert_of, nused_arr, xs, W1e, W2e)

    ys4 = pl.kernel(
        _combine_gather_body, mesh=mesh,
        out_type=jax.ShapeDtypeStruct((NASSIGN, D_MODEL), jnp.float32),
        scratch_types=[
            pltpu.VMEM((NCH, CH), jnp.int32),
            pltpu.VMEM((CH, D_MODEL), jnp.float32),
            pltpu.SemaphoreType.DMA,
        ],
    )(ys, idx3d)
    ys4r = ys4.reshape(TOP_K + 1, N_TOK, D_MODEL)

    TT = 512
    out = pl.pallas_call(
        _combine_body,
        grid=(N_TOK // TT,),
        in_specs=[
            pl.BlockSpec((TOP_K + 1, TT, D_MODEL), lambda t: (0, t, 0)),
            pl.BlockSpec((TOP_K + 1, TT), lambda t: (0, t)),
        ],
        out_specs=pl.BlockSpec((TT, D_MODEL), lambda t: (t, 0)),
        out_shape=jax.ShapeDtypeStruct((N_TOK, D_MODEL), jnp.float32),
    )(ys4r, w4)

    return out.reshape(B, T, D_MODEL)


# speedup vs baseline: 1.9014x; 1.0763x over previous
import functools

import jax
import jax.numpy as jnp
from jax import lax
from jax.experimental import pallas as pl
from jax.experimental.pallas import tpu as pltpu
from jax.experimental.pallas import tpu_sc as plsc

D_MODEL = 1024
HID = 1024
N_ROUTED = 15
TOP_K = 3
EPS = 1e-09
RMS_EPS = 1.1920929e-07

N_TOK = 2048
RT = 256
N_EXP = 16

TM = 512
G = (N_TOK * TOP_K) // TM + N_ROUTED
PMAX = G * TM

NASSIGN = N_TOK * TOP_K

SC_NC = 2
SC_NS = 16
SC_NW = SC_NC * SC_NS
RPW = NASSIGN // SC_NW
CH = 64
NCH = RPW // CH


def _router_body(x_ref, r_ref, xhat_ref, ti_ref, tw_ref):
    x = x_ref[...]
    v = jnp.mean(x * x, axis=-1, keepdims=True)
    xhat_ref[...] = x * jax.lax.rsqrt(v + RMS_EPS)
    logits = jax.lax.dot_general(x, r_ref[...], (((1,), (0,)), ((), ())),
                                 preferred_element_type=jnp.float32)
    m = jnp.max(logits, axis=-1, keepdims=True)
    eg = jnp.exp(logits - m)
    gates = eg / jnp.sum(eg, axis=-1, keepdims=True)
    lanes = jax.lax.broadcasted_iota(jnp.int32, (RT, N_ROUTED), 1)
    g = gates
    idxs, vals = [], []
    for _ in range(TOP_K):
        vals.append(jnp.max(g, axis=-1, keepdims=True))
        j = jnp.argmax(g, axis=-1)[:, None]
        idxs.append(j)
        g = jnp.where(lanes == j, -1.0, g)
    tot = vals[0] + vals[1] + vals[2] + EPS
    topw = jnp.concatenate(vals, axis=1) / tot
    topi = jnp.concatenate(idxs, axis=1)
    ti_ref[...] = jnp.concatenate(
        [topi, jnp.zeros((RT, N_EXP - TOP_K), jnp.int32)], axis=1)
    tw_ref[...] = jnp.concatenate(
        [topw, jnp.zeros((RT, N_EXP - TOP_K), jnp.float32)], axis=1)


def _dispatch_body(xhat_hbm, idx_hbm, xs_hbm, idx_v, rows_v, sem):
    wid = lax.axis_index("s") * SC_NC + lax.axis_index("c")
    pltpu.sync_copy(idx_hbm.at[wid], idx_v)
    base = wid * RPW
    for j in range(NCH):
        src = (base + j * CH) % N_TOK
        pltpu.sync_copy(xhat_hbm.at[pl.ds(src, CH)], rows_v)
        pltpu.async_copy(rows_v, xs_hbm.at[idx_v.at[j]], sem).wait()


def _combine_gather_body(ys_hbm, idx_hbm, ys4_hbm, idx_v, rows_v, sem):
    wid = lax.axis_index("s") * SC_NC + lax.axis_index("c")
    pltpu.sync_copy(idx_hbm.at[wid], idx_v)
    base = wid * RPW
    for j in range(NCH):
        pltpu.async_copy(ys_hbm.at[idx_v.at[j]], rows_v, sem).wait()
        pltpu.sync_copy(rows_v, ys4_hbm.at[pl.ds(base + j * CH, CH)])


def _ffn_body(expert_of_ref, nused_ref, xs_ref, W1_ref, W2_ref, ys_ref):
    g = pl.program_id(0)

    @pl.when(g < nused_ref[0])
    def _():
        xh = xs_ref[...].astype(jnp.bfloat16)
        h = jax.lax.dot_general(xh, W1_ref[0], (((1,), (1,)), ((), ())),
                                preferred_element_type=jnp.float32)
        h = h * jax.nn.sigmoid(h)
        y = jax.lax.dot_general(h.astype(jnp.bfloat16), W2_ref[0],
                                (((1,), (1,)), ((), ())),
                                preferred_element_type=jnp.float32)
        ys_ref[...] = y


def _ffn_shared_body(xhat_ref, W1_ref, W2_ref, ysh_ref):
    xh = xhat_ref[...].astype(jnp.bfloat16)
    h = jax.lax.dot_general(xh, W1_ref[0], (((1,), (1,)), ((), ())),
                            preferred_element_type=jnp.float32)
    h = h * jax.nn.sigmoid(h)
    ysh_ref[...] = jax.lax.dot_general(h.astype(jnp.bfloat16), W2_ref[0],
                                       (((1,), (1,)), ((), ())),
                                       preferred_element_type=jnp.float32)


def _combine_body(ys4_ref, w3_ref, ysh_ref, out_ref):
    acc = ysh_ref[...]
    for k in range(TOP_K):
        acc += ys4_ref[k] * w3_ref[k, :][:, None]
    out_ref[...] = acc


@jax.jit
def kernel(x, router, W1_r, W2_r, g_r, W1_s, W2_s, g_s):
    B, T, _ = x.shape
    xf = x.reshape(B * T, D_MODEL)
    W1e = jnp.concatenate([W1_r * g_r[:, None, :], W1_s * g_s[:, None, :]],
                          axis=0).astype(jnp.bfloat16)
    W2e = jnp.concatenate([W2_r, W2_s], axis=0).astype(jnp.bfloat16)

    xhat, ti16, tw16 = pl.pallas_call(
        _router_body,
        grid=(N_TOK // RT,),
        in_specs=[
            pl.BlockSpec((RT, D_MODEL), lambda t: (t, 0)),
            pl.BlockSpec((D_MODEL, N_ROUTED), lambda t: (0, 0)),
        ],
        out_specs=[
            pl.BlockSpec((RT, D_MODEL), lambda t: (t, 0)),
            pl.BlockSpec((RT, N_EXP), lambda t: (t, 0)),
            pl.BlockSpec((RT, N_EXP), lambda t: (t, 0)),
        ],
        out_shape=[
            jax.ShapeDtypeStruct((N_TOK, D_MODEL), jnp.float32),
            jax.ShapeDtypeStruct((N_TOK, N_EXP), jnp.int32),
            jax.ShapeDtypeStruct((N_TOK, N_EXP), jnp.float32),
        ],
    )(xf, router)
    top_i = ti16[:, :TOP_K]
    top_w = tw16[:, :TOP_K]

    e_lanes = jnp.arange(N_EXP, dtype=jnp.int32)
    onehot = (top_i[:, :, None] == e_lanes[None, None, :]).astype(jnp.int32)
    Xtok = onehot.sum(axis=1)
    Xc = jnp.cumsum(Xtok, axis=0)
    counts = Xc[-1]
    tiles_e = (counts + TM - 1) // TM
    cum_tiles = jnp.cumsum(tiles_e)
    tile_start = cum_tiles - tiles_e
    n_used = cum_tiles[-1]
    pstart = tile_start * TM
    Xex = Xc - Xtok
    rank = (onehot * Xex[:, None, :]).sum(-1)
    padpos = (onehot * pstart[None, None, :]).sum(-1) + rank
    padflat = jnp.concatenate(
        [padpos[:, 0], padpos[:, 1], padpos[:, 2]])
    idx3d = padflat.reshape(SC_NW, NCH, CH)
    w3 = top_w.T

    grange = jnp.arange(G, dtype=jnp.int32)
    expert_of = jnp.minimum(
        (grange[:, None] >= cum_tiles[None, :]).astype(jnp.int32).sum(-1),
        N_EXP - 2)
    nused_arr = n_used[None]

    mesh = plsc.VectorSubcoreMesh(core_axis_name="c", subcore_axis_name="s")
    xs = pl.kernel(
        _dispatch_body, mesh=mesh,
        out_type=jax.ShapeDtypeStruct((PMAX, D_MODEL), jnp.float32),
        scratch_types=[
            pltpu.VMEM((NCH, CH), jnp.int32),
            pltpu.VMEM((CH, D_MODEL), jnp.float32),
            pltpu.SemaphoreType.DMA,
        ],
    )(xhat, idx3d)

    ys = pl.pallas_call(
        _ffn_body,
        grid_spec=pltpu.PrefetchScalarGridSpec(
            num_scalar_prefetch=2,
            grid=(G,),
            in_specs=[
                pl.BlockSpec((TM, D_MODEL),
                             lambda g, eo, nu: (jnp.minimum(g, nu[0] - 1), 0)),
                pl.BlockSpec((1, HID, D_MODEL), lambda g, eo, nu: (eo[g], 0, 0)),
                pl.BlockSpec((1, D_MODEL, HID), lambda g, eo, nu: (eo[g], 0, 0)),
            ],
            out_specs=pl.BlockSpec(
                (TM, D_MODEL), lambda g, eo, nu: (jnp.minimum(g, nu[0] - 1), 0)),
        ),
        out_shape=jax.ShapeDtypeStruct((PMAX, D_MODEL), jnp.float32),
    )(expert_of, nused_arr, xs, W1e, W2e)

    ys4 = pl.kernel(
        _combine_gather_body, mesh=mesh,
        out_type=jax.ShapeDtypeStruct((NASSIGN, D_MODEL), jnp.float32),
        scratch_types=[
            pltpu.VMEM((NCH, CH), jnp.int32),
            pltpu.VMEM((CH, D_MODEL), jnp.float32),
            pltpu.SemaphoreType.DMA,
        ],
    )(ys, idx3d)
    ys4r = ys4.reshape(TOP_K, N_TOK, D_MODEL)

    ysh = pl.pallas_call(
        _ffn_shared_body,
        grid=(N_TOK // TM,),
        in_specs=[
            pl.BlockSpec((TM, D_MODEL), lambda t: (t, 0)),
            pl.BlockSpec((1, HID, D_MODEL), lambda t: (0, 0, 0)),
            pl.BlockSpec((1, D_MODEL, HID), lambda t: (0, 0, 0)),
        ],
        out_specs=pl.BlockSpec((TM, D_MODEL), lambda t: (t, 0)),
        out_shape=jax.ShapeDtypeStruct((N_TOK, D_MODEL), jnp.float32),
    )(xhat, W1e[N_EXP - 1:], W2e[N_EXP - 1:])

    TT = 512
    out = pl.pallas_call(
        _combine_body,
        grid=(N_TOK // TT,),
        in_specs=[
            pl.BlockSpec((TOP_K, TT, D_MODEL), lambda t: (0, t, 0)),
            pl.BlockSpec((TOP_K, TT), lambda t: (0, t)),
            pl.BlockSpec((TT, D_MODEL), lambda t: (t, 0)),
        ],
        out_specs=pl.BlockSpec((TT, D_MODEL), lambda t: (t, 0)),
        out_shape=jax.ShapeDtypeStruct((N_TOK, D_MODEL), jnp.float32),
    )(ys4r, w3, ysh)

    return out.reshape(B, T, D_MODEL)


# speedup vs baseline: 1.9037x; 1.0012x over previous
import functools

import jax
import jax.numpy as jnp
from jax import lax
from jax.experimental import pallas as pl
from jax.experimental.pallas import tpu as pltpu
from jax.experimental.pallas import tpu_sc as plsc

D_MODEL = 1024
HID = 1024
N_ROUTED = 15
TOP_K = 3
EPS = 1e-09
RMS_EPS = 1.1920929e-07

N_TOK = 2048
RT = 256
N_EXP = 16

TM = 512
G = (N_TOK * TOP_K) // TM + N_ROUTED
PMAX = G * TM

NASSIGN = N_TOK * TOP_K

SC_NC = 2
SC_NS = 16
SC_NW = SC_NC * SC_NS
RPW = NASSIGN // SC_NW
CH = 32
NCH = RPW // CH


def _router_body(x_ref, r_ref, xhat_ref, ti_ref, tw_ref):
    x = x_ref[...]
    v = jnp.mean(x * x, axis=-1, keepdims=True)
    xhat_ref[...] = x * jax.lax.rsqrt(v + RMS_EPS)
    logits = jax.lax.dot_general(x, r_ref[...], (((1,), (0,)), ((), ())),
                                 preferred_element_type=jnp.float32)
    m = jnp.max(logits, axis=-1, keepdims=True)
    eg = jnp.exp(logits - m)
    gates = eg / jnp.sum(eg, axis=-1, keepdims=True)
    lanes = jax.lax.broadcasted_iota(jnp.int32, (RT, N_ROUTED), 1)
    g = gates
    idxs, vals = [], []
    for _ in range(TOP_K):
        vals.append(jnp.max(g, axis=-1, keepdims=True))
        j = jnp.argmax(g, axis=-1)[:, None]
        idxs.append(j)
        g = jnp.where(lanes == j, -1.0, g)
    tot = vals[0] + vals[1] + vals[2] + EPS
    topw = jnp.concatenate(vals, axis=1) / tot
    topi = jnp.concatenate(idxs, axis=1)
    ti_ref[...] = jnp.concatenate(
        [topi, jnp.zeros((RT, N_EXP - TOP_K), jnp.int32)], axis=1)
    tw_ref[...] = jnp.concatenate(
        [topw, jnp.zeros((RT, N_EXP - TOP_K), jnp.float32)], axis=1)


def _dispatch_body(xhat_hbm, idx_hbm, xs_hbm, idx_v, rows_v, sem):
    wid = lax.axis_index("s") * SC_NC + lax.axis_index("c")
    pltpu.sync_copy(idx_hbm.at[wid], idx_v)
    base = wid * RPW
    scat = [None] * NCH
    for j in range(NCH):
        b = j % 2
        if j >= 2:
            scat[j - 2].wait()
        src = (base + j * CH) % N_TOK
        pltpu.sync_copy(xhat_hbm.at[pl.ds(src, CH)], rows_v.at[b])
        scat[j] = pltpu.async_copy(rows_v.at[b], xs_hbm.at[idx_v.at[j]], sem)
    scat[NCH - 2].wait()
    scat[NCH - 1].wait()


def _combine_gather_body(ys_hbm, idx_hbm, ys4_hbm, idx_v, rows_v, sem):
    wid = lax.axis_index("s") * SC_NC + lax.axis_index("c")
    pltpu.sync_copy(idx_hbm.at[wid], idx_v)
    base = wid * RPW
    gath = [None] * NCH
    gath[0] = pltpu.async_copy(ys_hbm.at[idx_v.at[0]], rows_v.at[0], sem)
    for j in range(NCH):
        if j + 1 < NCH:
            gath[j + 1] = pltpu.async_copy(
                ys_hbm.at[idx_v.at[j + 1]], rows_v.at[(j + 1) % 2], sem)
        gath[j].wait()
        pltpu.sync_copy(rows_v.at[j % 2],
                        ys4_hbm.at[pl.ds(base + j * CH, CH)])


def _ffn_body(expert_of_ref, nused_ref, xs_ref, W1_ref, W2_ref, ys_ref):
    g = pl.program_id(0)

    @pl.when(g < nused_ref[0])
    def _():
        xh = xs_ref[...].astype(jnp.bfloat16)
        h = jax.lax.dot_general(xh, W1_ref[0], (((1,), (1,)), ((), ())),
                                preferred_element_type=jnp.float32)
        h = h * jax.nn.sigmoid(h)
        y = jax.lax.dot_general(h.astype(jnp.bfloat16), W2_ref[0],
                                (((1,), (1,)), ((), ())),
                                preferred_element_type=jnp.float32)
        ys_ref[...] = y


def _ffn_shared_body(xhat_ref, W1_ref, W2_ref, ysh_ref):
    xh = xhat_ref[...].astype(jnp.bfloat16)
    h = jax.lax.dot_general(xh, W1_ref[0], (((1,), (1,)), ((), ())),
                            preferred_element_type=jnp.float32)
    h = h * jax.nn.sigmoid(h)
    ysh_ref[...] = jax.lax.dot_general(h.astype(jnp.bfloat16), W2_ref[0],
                                       (((1,), (1,)), ((), ())),
                                       preferred_element_type=jnp.float32)


def _combine_body(ys4_ref, w3_ref, ysh_ref, out_ref):
    acc = ysh_ref[...]
    for k in range(TOP_K):
        acc += ys4_ref[k] * w3_ref[k, :][:, None]
    out_ref[...] = acc


@jax.jit
def kernel(x, router, W1_r, W2_r, g_r, W1_s, W2_s, g_s):
    B, T, _ = x.shape
    xf = x.reshape(B * T, D_MODEL)
    W1e = jnp.concatenate([W1_r * g_r[:, None, :], W1_s * g_s[:, None, :]],
                          axis=0).astype(jnp.bfloat16)
    W2e = jnp.concatenate([W2_r, W2_s], axis=0).astype(jnp.bfloat16)

    xhat, ti16, tw16 = pl.pallas_call(
        _router_body,
        grid=(N_TOK // RT,),
        in_specs=[
            pl.BlockSpec((RT, D_MODEL), lambda t: (t, 0)),
            pl.BlockSpec((D_MODEL, N_ROUTED), lambda t: (0, 0)),
        ],
        out_specs=[
            pl.BlockSpec((RT, D_MODEL), lambda t: (t, 0)),
            pl.BlockSpec((RT, N_EXP), lambda t: (t, 0)),
            pl.BlockSpec((RT, N_EXP), lambda t: (t, 0)),
        ],
        out_shape=[
            jax.ShapeDtypeStruct((N_TOK, D_MODEL), jnp.float32),
            jax.ShapeDtypeStruct((N_TOK, N_EXP), jnp.int32),
            jax.ShapeDtypeStruct((N_TOK, N_EXP), jnp.float32),
        ],
    )(xf, router)
    top_i = ti16[:, :TOP_K]
    top_w = tw16[:, :TOP_K]

    e_lanes = jnp.arange(N_EXP, dtype=jnp.int32)
    onehot = (top_i[:, :, None] == e_lanes[None, None, :]).astype(jnp.int32)
    Xtok = onehot.sum(axis=1)
    Xc = jnp.cumsum(Xtok, axis=0)
    counts = Xc[-1]
    tiles_e = (counts + TM - 1) // TM
    cum_tiles = jnp.cumsum(tiles_e)
    tile_start = cum_tiles - tiles_e
    n_used = cum_tiles[-1]
    pstart = tile_start * TM
    Xex = Xc - Xtok
    rank = (onehot * Xex[:, None, :]).sum(-1)
    padpos = (onehot * pstart[None, None, :]).sum(-1) + rank
    padflat = jnp.concatenate(
        [padpos[:, 0], padpos[:, 1], padpos[:, 2]])
    idx3d = padflat.reshape(SC_NW, NCH, CH)
    w3 = top_w.T

    grange = jnp.arange(G, dtype=jnp.int32)
    expert_of = jnp.minimum(
        (grange[:, None] >= cum_tiles[None, :]).astype(jnp.int32).sum(-1),
        N_EXP - 2)
    nused_arr = n_used[None]

    mesh = plsc.VectorSubcoreMesh(core_axis_name="c", subcore_axis_name="s")
    xs = pl.kernel(
        _dispatch_body, mesh=mesh,
        out_type=jax.ShapeDtypeStruct((PMAX, D_MODEL), jnp.float32),
        scratch_types=[
            pltpu.VMEM((NCH, CH), jnp.int32),
            pltpu.VMEM((2, CH, D_MODEL), jnp.float32),
            pltpu.SemaphoreType.DMA,
        ],
    )(xhat, idx3d)

    ys = pl.pallas_call(
        _ffn_body,
        grid_spec=pltpu.PrefetchScalarGridSpec(
            num_scalar_prefetch=2,
            grid=(G,),
            in_specs=[
                pl.BlockSpec((TM, D_MODEL),
                             lambda g, eo, nu: (jnp.minimum(g, nu[0] - 1), 0)),
                pl.BlockSpec((1, HID, D_MODEL), lambda g, eo, nu: (eo[g], 0, 0)),
                pl.BlockSpec((1, D_MODEL, HID), lambda g, eo, nu: (eo[g], 0, 0)),
            ],
            out_specs=pl.BlockSpec(
                (TM, D_MODEL), lambda g, eo, nu: (jnp.minimum(g, nu[0] - 1), 0)),
        ),
        out_shape=jax.ShapeDtypeStruct((PMAX, D_MODEL), jnp.float32),
    )(expert_of, nused_arr, xs, W1e, W2e)

    ys4 = pl.kernel(
        _combine_gather_body, mesh=mesh,
        out_type=jax.ShapeDtypeStruct((NASSIGN, D_MODEL), jnp.float32),
        scratch_types=[
            pltpu.VMEM((NCH, CH), jnp.int32),
            pltpu.VMEM((2, CH, D_MODEL), jnp.float32),
            pltpu.SemaphoreType.DMA,
        ],
    )(ys, idx3d)
    ys4r = ys4.reshape(TOP_K, N_TOK, D_MODEL)

    ysh = pl.pallas_call(
        _ffn_shared_body,
        grid=(N_TOK // TM,),
        in_specs=[
            pl.BlockSpec((TM, D_MODEL), lambda t: (t, 0)),
            pl.BlockSpec((1, HID, D_MODEL), lambda t: (0, 0, 0)),
            pl.BlockSpec((1, D_MODEL, HID), lambda t: (0, 0, 0)),
        ],
        out_specs=pl.BlockSpec((TM, D_MODEL), lambda t: (t, 0)),
        out_shape=jax.ShapeDtypeStruct((N_TOK, D_MODEL), jnp.float32),
    )(xhat, W1e[N_EXP - 1:], W2e[N_EXP - 1:])

    TT = 512
    out = pl.pallas_call(
        _combine_body,
        grid=(N_TOK // TT,),
        in_specs=[
            pl.BlockSpec((TOP_K, TT, D_MODEL), lambda t: (0, t, 0)),
            pl.BlockSpec((TOP_K, TT), lambda t: (0, t)),
            pl.BlockSpec((TT, D_MODEL), lambda t: (t, 0)),
        ],
        out_specs=pl.BlockSpec((TT, D_MODEL), lambda t: (t, 0)),
        out_shape=jax.ShapeDtypeStruct((N_TOK, D_MODEL), jnp.float32),
    )(ys4r, w3, ysh)

    return out.reshape(B, T, D_MODEL)


# speedup vs baseline: 1.9075x; 1.0020x over previous
import jax
import jax.numpy as jnp
from jax import lax
from jax.experimental import pallas as pl
from jax.experimental.pallas import tpu as pltpu
from jax.experimental.pallas import tpu_sc as plsc

D_MODEL = 1024
HID = 1024
N_ROUTED = 15
TOP_K = 3
EPS = 1e-09
RMS_EPS = 1.1920929e-07

N_TOK = 2048
RT = 256
N_EXP = 16

TM = 512
G = (N_TOK * TOP_K) // TM + N_ROUTED
PMAX = G * TM

NASSIGN = N_TOK * TOP_K

SC_NC = 2
SC_NS = 16
SC_NW = SC_NC * SC_NS
RPW = NASSIGN // SC_NW
CH = 32
NCH = RPW // CH


def _router_body(x_ref, r_ref, xhat_ref, ti_ref, tw_ref):
    x = x_ref[...]
    v = jnp.mean(x * x, axis=-1, keepdims=True)
    xhat_ref[...] = x * jax.lax.rsqrt(v + RMS_EPS)
    logits = jax.lax.dot_general(x, r_ref[...], (((1,), (0,)), ((), ())),
                                 preferred_element_type=jnp.float32)
    m = jnp.max(logits, axis=-1, keepdims=True)
    eg = jnp.exp(logits - m)
    gates = eg / jnp.sum(eg, axis=-1, keepdims=True)
    lanes = jax.lax.broadcasted_iota(jnp.int32, (RT, N_ROUTED), 1)
    g = gates
    idxs, vals = [], []
    for _ in range(TOP_K):
        vals.append(jnp.max(g, axis=-1, keepdims=True))
        j = jnp.argmax(g, axis=-1)[:, None]
        idxs.append(j)
        g = jnp.where(lanes == j, -1.0, g)
    tot = vals[0] + vals[1] + vals[2] + EPS
    topw = jnp.concatenate(vals, axis=1) / tot
    topi = jnp.concatenate(idxs, axis=1)
    ti_ref[...] = jnp.concatenate(
        [topi, jnp.zeros((RT, N_EXP - TOP_K), jnp.int32)], axis=1)
    tw_ref[...] = jnp.concatenate(
        [topw, jnp.zeros((RT, N_EXP - TOP_K), jnp.float32)], axis=1)


def _dispatch_body(xhat_hbm, idx_hbm, xs_hbm, idx_v, rows_v, sem):
    wid = lax.axis_index("s") * SC_NC + lax.axis_index("c")
    pltpu.sync_copy(idx_hbm.at[wid], idx_v)
    base = wid * RPW
    scat = [None] * NCH
    for j in range(NCH):
        b = j % 2
        if j >= 2:
            scat[j - 2].wait()
        src = (base + j * CH) % N_TOK
        pltpu.sync_copy(xhat_hbm.at[pl.ds(src, CH)], rows_v.at[b])
        scat[j] = pltpu.async_copy(rows_v.at[b], xs_hbm.at[idx_v.at[j]], sem)
    scat[NCH - 2].wait()
    scat[NCH - 1].wait()


def _combine_gather_body(ys_hbm, idx_hbm, ys4_hbm, idx_v, rows_v, sem):
    wid = lax.axis_index("s") * SC_NC + lax.axis_index("c")
    pltpu.sync_copy(idx_hbm.at[wid], idx_v)
    base = wid * RPW
    gath = [None] * NCH
    gath[0] = pltpu.async_copy(ys_hbm.at[idx_v.at[0]], rows_v.at[0], sem)
    for j in range(NCH):
        if j + 1 < NCH:
            gath[j + 1] = pltpu.async_copy(
                ys_hbm.at[idx_v.at[j + 1]], rows_v.at[(j + 1) % 2], sem)
        gath[j].wait()
        pltpu.sync_copy(rows_v.at[j % 2],
                        ys4_hbm.at[pl.ds(base + j * CH, CH)])


def _ffn_body(expert_of_ref, nused_ref, xs_ref, W1_ref, W2_ref, ys_ref):
    g = pl.program_id(0)

    @pl.when(g < nused_ref[0])
    def _():
        xh = xs_ref[...].astype(jnp.bfloat16)
        h = jax.lax.dot_general(xh, W1_ref[0], (((1,), (1,)), ((), ())),
                                preferred_element_type=jnp.float32)
        h = h * jax.nn.sigmoid(h)
        y = jax.lax.dot_general(h.astype(jnp.bfloat16), W2_ref[0],
                                (((1,), (1,)), ((), ())),
                                preferred_element_type=jnp.float32)
        ys_ref[...] = y


def _ffn_shared_body(xhat_ref, W1_ref, W2_ref, ysh_ref):
    xh = xhat_ref[...].astype(jnp.bfloat16)
    h = jax.lax.dot_general(xh, W1_ref[0], (((1,), (1,)), ((), ())),
                            preferred_element_type=jnp.float32)
    h = h * jax.nn.sigmoid(h)
    ysh_ref[...] = jax.lax.dot_general(h.astype(jnp.bfloat16), W2_ref[0],
                                       (((1,), (1,)), ((), ())),
                                       preferred_element_type=jnp.float32)


def _combine_body(ys4_ref, w3_ref, ysh_ref, out_ref):
    acc = ysh_ref[...]
    for k in range(TOP_K):
        acc += ys4_ref[k] * w3_ref[k, :][:, None]
    out_ref[...] = acc


@jax.jit
def kernel(x, router, W1_r, W2_r, g_r, W1_s, W2_s, g_s):
    B, T, _ = x.shape
    xf = x.reshape(B * T, D_MODEL)
    W1e = jnp.concatenate([W1_r * g_r[:, None, :], W1_s * g_s[:, None, :]],
                          axis=0).astype(jnp.bfloat16)
    W2e = jnp.concatenate([W2_r, W2_s], axis=0).astype(jnp.bfloat16)

    xhat, ti16, tw16 = pl.pallas_call(
        _router_body,
        grid=(N_TOK // RT,),
        in_specs=[
            pl.BlockSpec((RT, D_MODEL), lambda t: (t, 0)),
            pl.BlockSpec((D_MODEL, N_ROUTED), lambda t: (0, 0)),
        ],
        out_specs=[
            pl.BlockSpec((RT, D_MODEL), lambda t: (t, 0)),
            pl.BlockSpec((RT, N_EXP), lambda t: (t, 0)),
            pl.BlockSpec((RT, N_EXP), lambda t: (t, 0)),
        ],
        out_shape=[
            jax.ShapeDtypeStruct((N_TOK, D_MODEL), jnp.float32),
            jax.ShapeDtypeStruct((N_TOK, N_EXP), jnp.int32),
            jax.ShapeDtypeStruct((N_TOK, N_EXP), jnp.float32),
        ],
    )(xf, router)
    top_i = ti16[:, :TOP_K]
    top_w = tw16[:, :TOP_K]

    e_lanes = jnp.arange(N_EXP, dtype=jnp.int32)
    onehot = (top_i[:, :, None] == e_lanes[None, None, :]).astype(jnp.int32)
    Xtok = onehot.sum(axis=1)
    Xc = jnp.cumsum(Xtok, axis=0)
    counts = Xc[-1]
    tiles_e = (counts + TM - 1) // TM
    cum_tiles = jnp.cumsum(tiles_e)
    tile_start = cum_tiles - tiles_e
    n_used = cum_tiles[-1]
    pstart = tile_start * TM
    Xex = Xc - Xtok
    rank = (onehot * Xex[:, None, :]).sum(-1)
    padpos = (onehot * pstart[None, None, :]).sum(-1) + rank
    padflat = jnp.concatenate(
        [padpos[:, 0], padpos[:, 1], padpos[:, 2]])
    idx3d = padflat.reshape(SC_NW, NCH, CH)
    w3 = top_w.T

    grange = jnp.arange(G, dtype=jnp.int32)
    expert_of = jnp.minimum(
        (grange[:, None] >= cum_tiles[None, :]).astype(jnp.int32).sum(-1),
        N_EXP - 2)
    nused_arr = n_used[None]

    mesh = plsc.VectorSubcoreMesh(core_axis_name="c", subcore_axis_name="s")
    xs = pl.kernel(
        _dispatch_body, mesh=mesh,
        out_type=jax.ShapeDtypeStruct((PMAX, D_MODEL), jnp.float32),
        scratch_types=[
            pltpu.VMEM((NCH, CH), jnp.int32),
            pltpu.VMEM((2, CH, D_MODEL), jnp.float32),
            pltpu.SemaphoreType.DMA,
        ],
    )(xhat, idx3d)

    ys = pl.pallas_call(
        _ffn_body,
        grid_spec=pltpu.PrefetchScalarGridSpec(
            num_scalar_prefetch=2,
            grid=(G,),
            in_specs=[
                pl.BlockSpec((TM, D_MODEL),
                             lambda g, eo, nu: (jnp.minimum(g, nu[0] - 1), 0)),
                pl.BlockSpec((1, HID, D_MODEL), lambda g, eo, nu: (eo[g], 0, 0)),
                pl.BlockSpec((1, D_MODEL, HID), lambda g, eo, nu: (eo[g], 0, 0)),
            ],
            out_specs=pl.BlockSpec(
                (TM, D_MODEL), lambda g, eo, nu: (jnp.minimum(g, nu[0] - 1), 0)),
        ),
        out_shape=jax.ShapeDtypeStruct((PMAX, D_MODEL), jnp.float32),
    )(expert_of, nused_arr, xs, W1e, W2e)

    ys4 = pl.kernel(
        _combine_gather_body, mesh=mesh,
        out_type=jax.ShapeDtypeStruct((NASSIGN, D_MODEL), jnp.float32),
        scratch_types=[
            pltpu.VMEM((NCH, CH), jnp.int32),
            pltpu.VMEM((2, CH, D_MODEL), jnp.float32),
            pltpu.SemaphoreType.DMA,
        ],
    )(ys, idx3d)
    ys4r = ys4.reshape(TOP_K, N_TOK, D_MODEL)

    ysh = pl.pallas_call(
        _ffn_shared_body,
        grid=(N_TOK // TM,),
        in_specs=[
            pl.BlockSpec((TM, D_MODEL), lambda t: (t, 0)),
            pl.BlockSpec((1, HID, D_MODEL), lambda t: (0, 0, 0)),
            pl.BlockSpec((1, D_MODEL, HID), lambda t: (0, 0, 0)),
        ],
        out_specs=pl.BlockSpec((TM, D_MODEL), lambda t: (t, 0)),
        out_shape=jax.ShapeDtypeStruct((N_TOK, D_MODEL), jnp.float32),
    )(xhat, W1e[N_EXP - 1:], W2e[N_EXP - 1:])

    TT = 512
    out = pl.pallas_call(
        _combine_body,
        grid=(N_TOK // TT,),
        in_specs=[
            pl.BlockSpec((TOP_K, TT, D_MODEL), lambda t: (0, t, 0)),
            pl.BlockSpec((TOP_K, TT), lambda t: (0, t)),
            pl.BlockSpec((TT, D_MODEL), lambda t: (t, 0)),
        ],
        out_specs=pl.BlockSpec((TT, D_MODEL), lambda t: (t, 0)),
        out_shape=jax.ShapeDtypeStruct((N_TOK, D_MODEL), jnp.float32),
    )(ys4r, w3, ysh)

    return out.reshape(B, T, D_MODEL)
